# Initial kernel scaffold; baseline (speedup 1.0000x reference)
import jax
import jax.numpy as jnp
from jax.experimental import pallas as pl


def kernel(input_, weight):
    raise NotImplementedError("write your pallas kernel here")


# speedup vs baseline: 3.0710x; 3.0710x over previous
import functools

import jax
import jax.numpy as jnp
from jax import lax
from jax.experimental import pallas as pl
from jax.experimental.pallas import tpu as pltpu
from jax.experimental.pallas import tpu_sc as plsc

NUM_EMBEDDINGS = 100000
EMBED_DIM = 128
BATCH = 4096
HIST_LEN = 50

_N = BATCH * HIST_LEN
_NC = 2
_NS = 16
_NW = _NC * _NS
_ROWS_PER_W = _N // _NW
_CH = 128
_K = _ROWS_PER_W // _CH


def _make_gather():
    mesh = plsc.VectorSubcoreMesh(core_axis_name="c", subcore_axis_name="s")

    @functools.partial(
        pl.kernel,
        mesh=mesh,
        out_type=jax.ShapeDtypeStruct((_N, EMBED_DIM), jnp.float32),
        scratch_types=[
            pltpu.VMEM((_K, _CH), jnp.int32),
            pltpu.VMEM((_CH, EMBED_DIM), jnp.float32),
            pltpu.SemaphoreType.DMA,
        ],
    )
    def gather_kernel(idx_hbm, table_hbm, out_hbm, idx_v, rows_v, sem):
        wid = lax.axis_index("s") * _NC + lax.axis_index("c")
        pltpu.sync_copy(idx_hbm.at[wid], idx_v)
        base = wid * _ROWS_PER_W

        def body(j, carry):
            pltpu.async_copy(table_hbm.at[idx_v.at[j]], rows_v, sem).wait()
            pltpu.sync_copy(rows_v, out_hbm.at[pl.ds(base + j * _CH, _CH)])
            return carry

        lax.fori_loop(0, _K, body, 0)

    return gather_kernel


_gather = _make_gather()


def kernel(input_, weight):
    idx = input_.reshape(_NW, _K, _CH).astype(jnp.int32)
    out = _gather(idx, weight)
    return out.reshape(BATCH, HIST_LEN, EMBED_DIM)


# speedup vs baseline: 3.4662x; 1.1287x over previous
import functools

import jax
import jax.numpy as jnp
from jax import lax
from jax.experimental import pallas as pl
from jax.experimental.pallas import tpu as pltpu
from jax.experimental.pallas import tpu_sc as plsc

NUM_EMBEDDINGS = 100000
EMBED_DIM = 128
BATCH = 4096
HIST_LEN = 50

_N = BATCH * HIST_LEN
_NC = 2
_NS = 16
_NW = _NC * _NS
_ROWS_PER_W = _N // _NW
_CH = 128
_K = _ROWS_PER_W // _CH


_NBUF = 5
_T = _K // _NBUF


def _make_gather():
    mesh = plsc.VectorSubcoreMesh(core_axis_name="c", subcore_axis_name="s")

    @functools.partial(
        pl.kernel,
        mesh=mesh,
        out_type=jax.ShapeDtypeStruct((_N, EMBED_DIM), jnp.float32),
        scratch_types=[
            pltpu.VMEM((_K, _CH), jnp.int32),
            pltpu.VMEM((_NBUF, _CH, EMBED_DIM), jnp.float32),
        ]
        + [pltpu.SemaphoreType.DMA] * _NBUF,
    )
    def gather_kernel(idx_hbm, table_hbm, out_hbm, idx_v, rows_v,
                      sem0, sem1, sem2, sem3, sem4):
        sems = [sem0, sem1, sem2, sem3, sem4]
        wid = lax.axis_index("s") * _NC + lax.axis_index("c")
        pltpu.sync_copy(idx_hbm.at[wid], idx_v)
        base = wid * _ROWS_PER_W

        def g_start(j, b):
            pltpu.async_copy(table_hbm.at[idx_v.at[j]], rows_v.at[b], sems[b])

        def g_wait(j, b):
            pltpu.make_async_copy(
                table_hbm.at[idx_v.at[j]], rows_v.at[b], sems[b]).wait()

        def out_slice(j):
            return out_hbm.at[pl.ds(base + j * _CH, _CH)]

        def s_start(j, b):
            pltpu.async_copy(rows_v.at[b], out_slice(j), sems[b])

        def s_wait(j, b):
            pltpu.make_async_copy(rows_v.at[b], out_slice(j), sems[b]).wait()

        for b in range(_NBUF):
            g_start(b, b)

        def outer(t, carry):
            for b in range(_NBUF):
                j = t * _NBUF + b
                g_wait(j, b)
                s_start(j, b)
                s_wait(j, b)
                g_start(j + _NBUF, b)
            return carry

        lax.fori_loop(0, _T - 1, outer, 0)

        for b in range(_NBUF):
            j = (_T - 1) * _NBUF + b
            g_wait(j, b)
            s_start(j, b)
            s_wait(j, b)

    return gather_kernel


_gather = _make_gather()


def kernel(input_, weight):
    idx = input_.reshape(_NW, _K, _CH).astype(jnp.int32)
    out = _gather(idx, weight)
    return out.reshape(BATCH, HIST_LEN, EMBED_DIM)
